# Initial kernel scaffold; baseline (speedup 1.0000x reference)
import jax
import jax.numpy as jnp
from jax.experimental import pallas as pl


def kernel(x, edge_index, edge_attr, params):
    raise NotImplementedError("write your pallas kernel here")


# speedup vs baseline: 3.4675x; 3.4675x over previous
import functools

import jax
import jax.numpy as jnp
from jax import lax
from jax.experimental import pallas as pl
from jax.experimental.pallas import tpu as pltpu
from jax.experimental.pallas import tpu_sc as plsc

N = 10000
E = 320000
NREL = 5
NC, NS = 2, 16
EB = 128
EPAD = 323584
EPT = EPAD // NS
NB1 = EPT // EB
EPT2 = EPAD // (NS * 2)
NB2 = EPT2 // EB
R1 = 51200
R2 = 10240
RPT = R1 // NS
RZ = 640
BLK = 2000
NBLK = N // BLK

_MESH = plsc.VectorSubcoreMesh(
    core_axis_name="c", subcore_axis_name="s", num_cores=NC, num_subcores=NS)
_f32 = jnp.float32


def _mm1(x_ref, w_ref, b_ref, xp_ref, xr_ref):
    t = jnp.dot(x_ref[...], w_ref[0], preferred_element_type=_f32) + b_ref[0]
    xp_ref[0] = t[:, :128]
    xr_ref[0] = t[:, 128:]


_mm1_call = pl.pallas_call(
    _mm1,
    grid=(NBLK, NREL),
    in_specs=[
        pl.BlockSpec((BLK, 128), lambda i, j: (i, 0)),
        pl.BlockSpec((1, 128, 256), lambda i, j: (j, 0, 0)),
        pl.BlockSpec((1, 1, 256), lambda i, j: (j, 0, 0)),
    ],
    out_specs=[
        pl.BlockSpec((1, BLK, 128), lambda i, j: (j, i, 0)),
        pl.BlockSpec((1, BLK, 128), lambda i, j: (j, i, 0)),
    ],
    out_shape=[
        jax.ShapeDtypeStruct((NREL, N, 128), _f32),
        jax.ShapeDtypeStruct((NREL, N, 128), _f32),
    ],
)


def _sc_edge1(xp_hbm, src_h, dst_h, attr_h, zeros_h, ones_h,
              agg_out, cnt_out,
              acc, es, ed, ea, ig, isx, gbuf, zbuf, dbuf, obuf):
    core = lax.axis_index("c")
    sub = lax.axis_index("s")
    pltpu.sync_copy(zeros_h, zbuf)
    pltpu.sync_copy(ones_h, obuf)
    row0 = sub * RPT

    def zero_acc():
        for z in range(RPT // RZ):
            pltpu.sync_copy(zbuf, acc.at[pl.ds(row0 + z * RZ, RZ)])

    ebase = sub * EPT
    for r in range(4):
        chunk = core * 4 + r
        zero_acc()
        plsc.subcore_barrier()

        def body(bi, carry):
            b = ebase + bi * EB
            pltpu.sync_copy(src_h.at[pl.ds(b, EB)], es)
            pltpu.sync_copy(dst_h.at[pl.ds(b, EB)], ed)
            pltpu.sync_copy(attr_h.at[pl.ds(b, EB)], ea)
            for i in range(EB // 16):
                sl = pl.ds(i * 16, 16)
                a16 = ea[sl]
                ig[sl] = a16 * (8 * N) + es[sl] * 8 + chunk
                isx[sl] = a16 * N + ed[sl]
            pltpu.sync_copy(xp_hbm.at[ig], gbuf)
            pltpu.sync_copy(gbuf, acc.at[isx], add=True)
            return carry

        lax.fori_loop(0, NB1, body, 0)
        plsc.subcore_barrier()
        for z in range(RPT // RZ):
            rr = row0 + z * RZ
            pltpu.sync_copy(acc.at[pl.ds(rr, RZ)], dbuf)
            pltpu.sync_copy(dbuf, agg_out.at[chunk, pl.ds(rr, RZ)])
        plsc.subcore_barrier()

    zero_acc()
    plsc.subcore_barrier()
    cbase = core * (EPAD // 2) + sub * EPT2

    def cbody(bi, carry):
        b = cbase + bi * EB
        pltpu.sync_copy(dst_h.at[pl.ds(b, EB)], ed)
        pltpu.sync_copy(attr_h.at[pl.ds(b, EB)], ea)
        for i in range(EB // 16):
            sl = pl.ds(i * 16, 16)
            isx[sl] = ea[sl] * N + ed[sl]
        pltpu.sync_copy(obuf, acc.at[isx], add=True)
        return carry

    lax.fori_loop(0, NB2, cbody, 0)
    plsc.subcore_barrier()
    for z in range(RPT // RZ):
        rr = row0 + z * RZ
        pltpu.sync_copy(acc.at[pl.ds(rr, RZ)], dbuf)
        pltpu.sync_copy(dbuf, cnt_out.at[core, pl.ds(rr, RZ)])


_edge1_call = pl.kernel(
    _sc_edge1,
    out_type=[
        jax.ShapeDtypeStruct((8, R1, 16), _f32),
        jax.ShapeDtypeStruct((2, R1, 16), _f32),
    ],
    mesh=_MESH,
    compiler_params=pltpu.CompilerParams(use_tc_tiling_on_sc=False),
    scratch_types=[
        pltpu.VMEM_SHARED((R1, 16), _f32),
        pltpu.VMEM((EB,), jnp.int32),
        pltpu.VMEM((EB,), jnp.int32),
        pltpu.VMEM((EB,), jnp.int32),
        pltpu.VMEM((EB,), jnp.int32),
        pltpu.VMEM((EB,), jnp.int32),
        pltpu.VMEM((EB, 16), _f32),
        pltpu.VMEM((RZ, 16), _f32),
        pltpu.VMEM((RZ, 16), _f32),
        pltpu.VMEM((EB, 16), _f32),
    ],
)


def _mid(a_ref, c0_ref, c1_ref, xr_ref, sc_ref, sh_ref, wl_ref, wr_ref, b2_ref,
         hp_ref, hr_ref, c2_ref):
    j = pl.program_id(1)
    c = c0_ref[0] + c1_ref[0]
    mean = a_ref[0] / jnp.maximum(c[:, :1], 1.0)
    t = jnp.maximum(mean + xr_ref[0], 0.0)
    hb = t * sc_ref[0] + sh_ref[0]
    pj = jnp.dot(hb, wl_ref[0], preferred_element_type=_f32)
    rj = jnp.dot(hb, wr_ref[0], preferred_element_type=_f32)

    @pl.when(j == 0)
    def _():
        hp_ref[...] = pj
        hr_ref[...] = rj + b2_ref[...]
        c2_ref[...] = c

    @pl.when(j != 0)
    def _():
        hp_ref[...] += pj
        hr_ref[...] += rj
        c2_ref[...] += c


_mid_call = pl.pallas_call(
    _mid,
    grid=(NBLK, NREL),
    in_specs=[
        pl.BlockSpec((1, BLK, 128), lambda i, j: (j, i, 0)),
        pl.BlockSpec((1, BLK, 16), lambda i, j: (j, i, 0)),
        pl.BlockSpec((1, BLK, 16), lambda i, j: (j, i, 0)),
        pl.BlockSpec((1, BLK, 128), lambda i, j: (j, i, 0)),
        pl.BlockSpec((1, 1, 128), lambda i, j: (j, 0, 0)),
        pl.BlockSpec((1, 1, 128), lambda i, j: (j, 0, 0)),
        pl.BlockSpec((1, 128, 128), lambda i, j: (j, 0, 0)),
        pl.BlockSpec((1, 128, 128), lambda i, j: (j, 0, 0)),
        pl.BlockSpec((1, 128), lambda i, j: (0, 0)),
    ],
    out_specs=[
        pl.BlockSpec((BLK, 128), lambda i, j: (i, 0)),
        pl.BlockSpec((BLK, 128), lambda i, j: (i, 0)),
        pl.BlockSpec((BLK, 16), lambda i, j: (i, 0)),
    ],
    out_shape=[
        jax.ShapeDtypeStruct((N, 128), _f32),
        jax.ShapeDtypeStruct((N, 128), _f32),
        jax.ShapeDtypeStruct((N, 16), _f32),
    ],
)


def _sc_edge2(hp_hbm, src_h, dst_h, zeros_h, agg_out,
              acc, es, ed, ig, gbuf, zbuf, dbuf):
    core = lax.axis_index("c")
    sub = lax.axis_index("s")
    pltpu.sync_copy(zeros_h, zbuf)
    row0 = sub * (R2 // NS)
    ebase = sub * EPT
    for r in range(2):
        chunk = core * 2 + r
        pltpu.sync_copy(zbuf, acc.at[pl.ds(row0, RZ)])
        plsc.subcore_barrier()

        def body(bi, carry):
            b = ebase + bi * EB
            pltpu.sync_copy(src_h.at[pl.ds(b, EB)], es)
            pltpu.sync_copy(dst_h.at[pl.ds(b, EB)], ed)
            for i in range(EB // 16):
                sl = pl.ds(i * 16, 16)
                ig[sl] = es[sl] * 4 + chunk
            pltpu.sync_copy(hp_hbm.at[ig], gbuf)
            pltpu.sync_copy(gbuf, acc.at[ed], add=True)
            return carry

        lax.fori_loop(0, NB1, body, 0)
        plsc.subcore_barrier()
        pltpu.sync_copy(acc.at[pl.ds(row0, RZ)], dbuf)
        pltpu.sync_copy(dbuf, agg_out.at[chunk, pl.ds(row0, RZ)])
        plsc.subcore_barrier()


_edge2_call = pl.kernel(
    _sc_edge2,
    out_type=jax.ShapeDtypeStruct((4, R2, 32), _f32),
    mesh=_MESH,
    compiler_params=pltpu.CompilerParams(use_tc_tiling_on_sc=False),
    scratch_types=[
        pltpu.VMEM_SHARED((R2, 32), _f32),
        pltpu.VMEM((EB,), jnp.int32),
        pltpu.VMEM((EB,), jnp.int32),
        pltpu.VMEM((EB,), jnp.int32),
        pltpu.VMEM((EB, 32), _f32),
        pltpu.VMEM((RZ, 32), _f32),
        pltpu.VMEM((RZ, 32), _f32),
    ],
)


def _fin(a_ref, c_ref, hr_ref, o_ref):
    o_ref[...] = a_ref[...] / jnp.maximum(c_ref[:, :1], 1.0) + hr_ref[...]


_fin_call = pl.pallas_call(
    _fin,
    grid=(NBLK,),
    in_specs=[
        pl.BlockSpec((BLK, 128), lambda i: (i, 0)),
        pl.BlockSpec((BLK, 16), lambda i: (i, 0)),
        pl.BlockSpec((BLK, 128), lambda i: (i, 0)),
    ],
    out_specs=pl.BlockSpec((BLK, 128), lambda i: (i, 0)),
    out_shape=jax.ShapeDtypeStruct((N, 128), _f32),
)


def kernel(x, edge_index, edge_attr, params):
    names = ('gd', 'gg', 'bg', 'gp', 'dp')
    Wl = jnp.concatenate([params[n]['Wl'] for n in names], 1)
    Wr = jnp.concatenate([params[n]['Wr'] for n in names], 1)
    b = jnp.concatenate([params[n]['b'] for n in names], 0)
    Wl5 = Wl.reshape(128, NREL, 128).transpose(1, 0, 2)
    Wr5 = Wr.reshape(128, NREL, 128).transpose(1, 0, 2)
    Wcat = jnp.concatenate([Wl5, Wr5], axis=2)
    bias = jnp.concatenate(
        [jnp.zeros((NREL, 1, 128), _f32), b.reshape(NREL, 1, 128)], axis=2)

    src = edge_index[0].astype(jnp.int32)
    dst = edge_index[1].astype(jnp.int32)
    attr = edge_attr.astype(jnp.int32)
    pad = EPAD - E
    srcP = jnp.concatenate([src, jnp.full((pad,), N - 1, jnp.int32)])
    dstP = jnp.concatenate([dst, jnp.full((pad,), N, jnp.int32)])
    attrP = jnp.concatenate([attr, jnp.full((pad,), NREL - 1, jnp.int32)])

    xp_t, xr_t = _mm1_call(x, Wcat, bias)
    xp_flat = xp_t.reshape(NREL * N * 8, 16)

    zeros16 = jnp.zeros((RZ, 16), _f32)
    ones16 = jnp.ones((EB, 16), _f32)
    zeros32 = jnp.zeros((RZ, 32), _f32)
    agg1c, cntc = _edge1_call(xp_flat, srcP, dstP, attrP, zeros16, ones16)
    agg1 = agg1c[:, :NREL * N].transpose(1, 0, 2).reshape(NREL, N, 128)
    cnt0 = cntc[0, :NREL * N].reshape(NREL, N, 16)
    cnt1 = cntc[1, :NREL * N].reshape(NREL, N, 16)

    bn = params['bn']
    scale = (bn['gamma'] / jnp.sqrt(bn['var'] + 1e-5))
    shift = bn['beta'] - bn['mean'] * scale
    scale5 = scale.reshape(NREL, 1, 128)
    shift5 = shift.reshape(NREL, 1, 128)
    Wl2 = jnp.concatenate(
        [params['mu']['Wl'], params['logvar']['Wl']], 1).reshape(NREL, 128, 128)
    Wr2 = jnp.concatenate(
        [params['mu']['Wr'], params['logvar']['Wr']], 1).reshape(NREL, 128, 128)
    b2 = jnp.concatenate(
        [params['mu']['b'], params['logvar']['b']], 0).reshape(1, 128)

    hp, hr, cnt2 = _mid_call(agg1, cnt0, cnt1, xr_t, scale5, shift5,
                             Wl2, Wr2, b2)
    hp_flat = hp.reshape(4 * N, 32)

    agg2c = _edge2_call(hp_flat, srcP, dstP, zeros32)
    agg2 = agg2c[:, :N].transpose(1, 0, 2).reshape(N, 128)

    out = _fin_call(agg2, cnt2, hr)
    return out[:, :64], out[:, 64:]


# speedup vs baseline: 6.4945x; 1.8729x over previous
import functools

import jax
import jax.numpy as jnp
from jax import lax
from jax.experimental import pallas as pl
from jax.experimental.pallas import tpu as pltpu
from jax.experimental.pallas import tpu_sc as plsc

N = 10000
E = 320000
NREL = 5
NC, NS = 2, 16
EB = 128
SUP1 = 8
SUP2 = 4
EPAD = 327680
EPT = EPAD // NS
FILL1 = SUP1 * EB
FILL2 = SUP2 * EB
NP1 = EPT // (2 * FILL1)
NP2B = EPT // (2 * FILL2)
EPT2 = EPAD // (NS * 2)
NPC = EPT2 // (2 * FILL1)
R1 = 51200
R2 = 10240
RPT = R1 // NS
RZ = 640
BLK = 2000
NBLK = N // BLK

_MESH = plsc.VectorSubcoreMesh(
    core_axis_name="c", subcore_axis_name="s", num_cores=NC, num_subcores=NS)
_f32 = jnp.float32


def _mm1(x_ref, w_ref, b_ref, xp_ref, xr_ref):
    t = jnp.dot(x_ref[...], w_ref[0], preferred_element_type=_f32) + b_ref[0]
    xp_ref[0] = t[:, :128]
    xr_ref[0] = t[:, 128:]


_mm1_call = pl.pallas_call(
    _mm1,
    grid=(NBLK, NREL),
    in_specs=[
        pl.BlockSpec((BLK, 128), lambda i, j: (i, 0)),
        pl.BlockSpec((1, 128, 256), lambda i, j: (j, 0, 0)),
        pl.BlockSpec((1, 1, 256), lambda i, j: (j, 0, 0)),
    ],
    out_specs=[
        pl.BlockSpec((1, BLK, 128), lambda i, j: (j, i, 0)),
        pl.BlockSpec((1, BLK, 128), lambda i, j: (j, i, 0)),
    ],
    out_shape=[
        jax.ShapeDtypeStruct((NREL, N, 128), _f32),
        jax.ShapeDtypeStruct((NREL, N, 128), _f32),
    ],
)


def _sc_edge1(xp_hbm, src_h, dst_h, attr_h, zeros_h, ones_h,
              agg_out, cnt_out,
              acc, es, ed, ea, ig, isx, gbuf, zbuf, dbuf, obuf,
              sem_i, sem_g, sem_s):
    core = lax.axis_index("c")
    sub = lax.axis_index("s")
    pltpu.sync_copy(zeros_h, zbuf)
    pltpu.sync_copy(ones_h, obuf)
    row0 = sub * RPT

    def zero_acc():
        for z in range(RPT // RZ):
            pltpu.sync_copy(zbuf, acc.at[pl.ds(row0 + z * RZ, RZ)])

    def dump_to(out_slab):
        for z in range(RPT // RZ):
            rr = row0 + z * RZ
            pltpu.sync_copy(acc.at[pl.ds(rr, RZ)], dbuf)
            pltpu.sync_copy(dbuf, out_slab.at[pl.ds(rr, RZ)])

    def make_pair_body(chunk, base, feature):
        def pair_body(t, carry):
            ldescs = []
            for h in range(2):
                b = base + (2 * t + h) * FILL1
                ldescs.append(pltpu.async_copy(
                    dst_h.at[pl.ds(b, FILL1)], ed.at[h], sem_i))
                ldescs.append(pltpu.async_copy(
                    attr_h.at[pl.ds(b, FILL1)], ea.at[h], sem_i))
                if feature:
                    ldescs.append(pltpu.async_copy(
                        src_h.at[pl.ds(b, FILL1)], es.at[h], sem_i))
            for d in ldescs:
                d.wait()
            for h in range(2):
                for j in range(SUP1):
                    for i in range(EB // 16):
                        sl = pl.ds(i * 16, 16)
                        slf = pl.ds(j * EB + i * 16, 16)
                        a16 = ea[h, slf]
                        if feature:
                            ig[h, j, sl] = a16 * (8 * N) + es[h, slf] * 8 + chunk
                        isx[h, j, sl] = a16 * N + ed[h, slf]
            if feature:
                gd = []
                for h in range(2):
                    for j in range(SUP1):
                        gd.append(pltpu.async_copy(
                            xp_hbm.at[ig.at[h, j]], gbuf.at[h, j], sem_g))
                sd = []
                for d in gd[:SUP1]:
                    d.wait()
                for j in range(SUP1):
                    sd.append(pltpu.async_copy(
                        gbuf.at[0, j], acc.at[isx.at[0, j]], sem_s, add=True))
                for d in gd[SUP1:]:
                    d.wait()
                for j in range(SUP1):
                    sd.append(pltpu.async_copy(
                        gbuf.at[1, j], acc.at[isx.at[1, j]], sem_s, add=True))
            else:
                sd = []
                for h in range(2):
                    for j in range(SUP1):
                        sd.append(pltpu.async_copy(
                            obuf, acc.at[isx.at[h, j]], sem_s, add=True))
            for d in sd:
                d.wait()
            return carry
        return pair_body

    ebase = sub * EPT
    for r in range(4):
        chunk = core * 4 + r
        zero_acc()
        plsc.subcore_barrier()
        lax.fori_loop(0, NP1, make_pair_body(chunk, ebase, True), 0)
        plsc.subcore_barrier()
        dump_to(agg_out.at[chunk])
        plsc.subcore_barrier()

    zero_acc()
    plsc.subcore_barrier()
    cbase = core * (EPAD // 2) + sub * EPT2
    lax.fori_loop(0, NPC, make_pair_body(0, cbase, False), 0)
    plsc.subcore_barrier()
    dump_to(cnt_out.at[core])


_edge1_call = pl.kernel(
    _sc_edge1,
    out_type=[
        jax.ShapeDtypeStruct((8, R1, 16), _f32),
        jax.ShapeDtypeStruct((2, R1, 16), _f32),
    ],
    mesh=_MESH,
    compiler_params=pltpu.CompilerParams(use_tc_tiling_on_sc=False),
    scratch_types=[
        pltpu.VMEM_SHARED((R1, 16), _f32),
        pltpu.VMEM((2, FILL1), jnp.int32),
        pltpu.VMEM((2, FILL1), jnp.int32),
        pltpu.VMEM((2, FILL1), jnp.int32),
        pltpu.VMEM((2, SUP1, EB), jnp.int32),
        pltpu.VMEM((2, SUP1, EB), jnp.int32),
        pltpu.VMEM((2, SUP1, EB, 16), _f32),
        pltpu.VMEM((RZ, 16), _f32),
        pltpu.VMEM((RZ, 16), _f32),
        pltpu.VMEM((EB, 16), _f32),
        pltpu.SemaphoreType.DMA,
        pltpu.SemaphoreType.DMA,
        pltpu.SemaphoreType.DMA,
    ],
)


def _mid(a_ref, c0_ref, c1_ref, xr_ref, sc_ref, sh_ref, wl_ref, wr_ref, b2_ref,
         hp_ref, hr_ref, c2_ref):
    j = pl.program_id(1)
    c = c0_ref[0] + c1_ref[0]
    mean = a_ref[0] / jnp.maximum(c[:, :1], 1.0)
    t = jnp.maximum(mean + xr_ref[0], 0.0)
    hb = t * sc_ref[0] + sh_ref[0]
    pj = jnp.dot(hb, wl_ref[0], preferred_element_type=_f32)
    rj = jnp.dot(hb, wr_ref[0], preferred_element_type=_f32)

    @pl.when(j == 0)
    def _():
        hp_ref[...] = pj
        hr_ref[...] = rj + b2_ref[...]
        c2_ref[...] = c

    @pl.when(j != 0)
    def _():
        hp_ref[...] += pj
        hr_ref[...] += rj
        c2_ref[...] += c


_mid_call = pl.pallas_call(
    _mid,
    grid=(NBLK, NREL),
    in_specs=[
        pl.BlockSpec((1, BLK, 128), lambda i, j: (j, i, 0)),
        pl.BlockSpec((1, BLK, 16), lambda i, j: (j, i, 0)),
        pl.BlockSpec((1, BLK, 16), lambda i, j: (j, i, 0)),
        pl.BlockSpec((1, BLK, 128), lambda i, j: (j, i, 0)),
        pl.BlockSpec((1, 1, 128), lambda i, j: (j, 0, 0)),
        pl.BlockSpec((1, 1, 128), lambda i, j: (j, 0, 0)),
        pl.BlockSpec((1, 128, 128), lambda i, j: (j, 0, 0)),
        pl.BlockSpec((1, 128, 128), lambda i, j: (j, 0, 0)),
        pl.BlockSpec((1, 128), lambda i, j: (0, 0)),
    ],
    out_specs=[
        pl.BlockSpec((BLK, 128), lambda i, j: (i, 0)),
        pl.BlockSpec((BLK, 128), lambda i, j: (i, 0)),
        pl.BlockSpec((BLK, 16), lambda i, j: (i, 0)),
    ],
    out_shape=[
        jax.ShapeDtypeStruct((N, 128), _f32),
        jax.ShapeDtypeStruct((N, 128), _f32),
        jax.ShapeDtypeStruct((N, 16), _f32),
    ],
)


def _sc_edge2(hp_hbm, src_h, dst_h, zeros_h, agg_out,
              acc, es, ed, ig, isx, gbuf, zbuf, dbuf,
              sem_i, sem_g, sem_s):
    core = lax.axis_index("c")
    sub = lax.axis_index("s")
    pltpu.sync_copy(zeros_h, zbuf)
    row0 = sub * (R2 // NS)
    ebase = sub * EPT
    for r in range(2):
        chunk = core * 2 + r
        pltpu.sync_copy(zbuf, acc.at[pl.ds(row0, RZ)])
        plsc.subcore_barrier()

        def pair_body(t, carry):
            ldescs = []
            for h in range(2):
                b = ebase + (2 * t + h) * FILL2
                ldescs.append(pltpu.async_copy(
                    src_h.at[pl.ds(b, FILL2)], es.at[h], sem_i))
                ldescs.append(pltpu.async_copy(
                    dst_h.at[pl.ds(b, FILL2)], ed.at[h], sem_i))
            for d in ldescs:
                d.wait()
            for h in range(2):
                for j in range(SUP2):
                    for i in range(EB // 16):
                        sl = pl.ds(i * 16, 16)
                        slf = pl.ds(j * EB + i * 16, 16)
                        ig[h, j, sl] = es[h, slf] * 4 + chunk
                        isx[h, j, sl] = ed[h, slf]
            gd = []
            for h in range(2):
                for j in range(SUP2):
                    gd.append(pltpu.async_copy(
                        hp_hbm.at[ig.at[h, j]], gbuf.at[h, j], sem_g))
            sd = []
            for d in gd[:SUP2]:
                d.wait()
            for j in range(SUP2):
                sd.append(pltpu.async_copy(
                    gbuf.at[0, j], acc.at[isx.at[0, j]], sem_s, add=True))
            for d in gd[SUP2:]:
                d.wait()
            for j in range(SUP2):
                sd.append(pltpu.async_copy(
                    gbuf.at[1, j], acc.at[isx.at[1, j]], sem_s, add=True))
            for d in sd:
                d.wait()
            return carry

        lax.fori_loop(0, NP2B, pair_body, 0)
        plsc.subcore_barrier()
        pltpu.sync_copy(acc.at[pl.ds(row0, RZ)], dbuf)
        pltpu.sync_copy(dbuf, agg_out.at[chunk, pl.ds(row0, RZ)])
        plsc.subcore_barrier()


_edge2_call = pl.kernel(
    _sc_edge2,
    out_type=jax.ShapeDtypeStruct((4, R2, 32), _f32),
    mesh=_MESH,
    compiler_params=pltpu.CompilerParams(use_tc_tiling_on_sc=False),
    scratch_types=[
        pltpu.VMEM_SHARED((R2, 32), _f32),
        pltpu.VMEM((2, FILL2), jnp.int32),
        pltpu.VMEM((2, FILL2), jnp.int32),
        pltpu.VMEM((2, SUP2, EB), jnp.int32),
        pltpu.VMEM((2, SUP2, EB), jnp.int32),
        pltpu.VMEM((2, SUP2, EB, 32), _f32),
        pltpu.VMEM((RZ, 32), _f32),
        pltpu.VMEM((RZ, 32), _f32),
        pltpu.SemaphoreType.DMA,
        pltpu.SemaphoreType.DMA,
        pltpu.SemaphoreType.DMA,
    ],
)


def _fin(a_ref, c_ref, hr_ref, o_ref):
    o_ref[...] = a_ref[...] / jnp.maximum(c_ref[:, :1], 1.0) + hr_ref[...]


_fin_call = pl.pallas_call(
    _fin,
    grid=(NBLK,),
    in_specs=[
        pl.BlockSpec((BLK, 128), lambda i: (i, 0)),
        pl.BlockSpec((BLK, 16), lambda i: (i, 0)),
        pl.BlockSpec((BLK, 128), lambda i: (i, 0)),
    ],
    out_specs=pl.BlockSpec((BLK, 128), lambda i: (i, 0)),
    out_shape=jax.ShapeDtypeStruct((N, 128), _f32),
)


def kernel(x, edge_index, edge_attr, params):
    names = ('gd', 'gg', 'bg', 'gp', 'dp')
    Wl = jnp.concatenate([params[n]['Wl'] for n in names], 1)
    Wr = jnp.concatenate([params[n]['Wr'] for n in names], 1)
    b = jnp.concatenate([params[n]['b'] for n in names], 0)
    Wl5 = Wl.reshape(128, NREL, 128).transpose(1, 0, 2)
    Wr5 = Wr.reshape(128, NREL, 128).transpose(1, 0, 2)
    Wcat = jnp.concatenate([Wl5, Wr5], axis=2)
    bias = jnp.concatenate(
        [jnp.zeros((NREL, 1, 128), _f32), b.reshape(NREL, 1, 128)], axis=2)

    src = edge_index[0].astype(jnp.int32)
    dst = edge_index[1].astype(jnp.int32)
    attr = edge_attr.astype(jnp.int32)
    pad = EPAD - E
    srcP = jnp.concatenate([src, jnp.full((pad,), N - 1, jnp.int32)])
    dstP = jnp.concatenate([dst, jnp.full((pad,), N, jnp.int32)])
    attrP = jnp.concatenate([attr, jnp.full((pad,), NREL - 1, jnp.int32)])

    xp_t, xr_t = _mm1_call(x, Wcat, bias)
    xp_flat = xp_t.reshape(NREL * N * 8, 16)

    zeros16 = jnp.zeros((RZ, 16), _f32)
    ones16 = jnp.ones((EB, 16), _f32)
    zeros32 = jnp.zeros((RZ, 32), _f32)
    agg1c, cntc = _edge1_call(xp_flat, srcP, dstP, attrP, zeros16, ones16)
    agg1 = agg1c[:, :NREL * N].transpose(1, 0, 2).reshape(NREL, N, 128)
    cnt0 = cntc[0, :NREL * N].reshape(NREL, N, 16)
    cnt1 = cntc[1, :NREL * N].reshape(NREL, N, 16)

    bn = params['bn']
    scale = (bn['gamma'] / jnp.sqrt(bn['var'] + 1e-5))
    shift = bn['beta'] - bn['mean'] * scale
    scale5 = scale.reshape(NREL, 1, 128)
    shift5 = shift.reshape(NREL, 1, 128)
    Wl2 = jnp.concatenate(
        [params['mu']['Wl'], params['logvar']['Wl']], 1).reshape(NREL, 128, 128)
    Wr2 = jnp.concatenate(
        [params['mu']['Wr'], params['logvar']['Wr']], 1).reshape(NREL, 128, 128)
    b2 = jnp.concatenate(
        [params['mu']['b'], params['logvar']['b']], 0).reshape(1, 128)

    hp, hr, cnt2 = _mid_call(agg1, cnt0, cnt1, xr_t, scale5, shift5,
                             Wl2, Wr2, b2)
    hp_flat = hp.reshape(4 * N, 32)

    agg2c = _edge2_call(hp_flat, srcP, dstP, zeros32)
    agg2 = agg2c[:, :N].transpose(1, 0, 2).reshape(N, 128)

    out = _fin_call(agg2, cnt2, hr)
    return out[:, :64], out[:, 64:]


# speedup vs baseline: 7.5031x; 1.1553x over previous
import functools

import jax
import jax.numpy as jnp
from jax import lax
from jax.experimental import pallas as pl
from jax.experimental.pallas import tpu as pltpu
from jax.experimental.pallas import tpu_sc as plsc

N = 10000
E = 320000
NREL = 5
NC, NS = 2, 16
EB = 128
SUP1 = 8
SUP2 = 4
EPAD = 327680
EPT = EPAD // NS
FILL1 = SUP1 * EB
FILL2 = SUP2 * EB
NP1 = EPT // (2 * FILL1)
NP2B = EPT // (2 * FILL2)
EPT2 = EPAD // (NS * 2)
NPC = EPT2 // (2 * FILL1)
NP = 10240
R1 = 51200
R2 = 10240
RPT = R1 // NS
RZ = 640
BLK = 2000
NBLK = N // BLK

_MESH = plsc.VectorSubcoreMesh(
    core_axis_name="c", subcore_axis_name="s", num_cores=NC, num_subcores=NS)
_f32 = jnp.float32


def _mm1(x_ref, w_ref, b_ref, xp_ref, xr_ref):
    t = jnp.dot(x_ref[...], w_ref[0], preferred_element_type=_f32) + b_ref[0]
    xp_ref[0] = t[:, :128]
    xr_ref[0] = t[:, 128:]


_mm1_call = pl.pallas_call(
    _mm1,
    grid=(NBLK, NREL),
    in_specs=[
        pl.BlockSpec((BLK, 128), lambda i, j: (i, 0)),
        pl.BlockSpec((1, 128, 256), lambda i, j: (j, 0, 0)),
        pl.BlockSpec((1, 1, 256), lambda i, j: (j, 0, 0)),
    ],
    out_specs=[
        pl.BlockSpec((1, BLK, 128), lambda i, j: (j, i, 0)),
        pl.BlockSpec((1, BLK, 128), lambda i, j: (j, i, 0)),
    ],
    out_shape=[
        jax.ShapeDtypeStruct((NREL, N, 128), _f32),
        jax.ShapeDtypeStruct((NREL, N, 128), _f32),
    ],
)


def _sc_edge1(xp_hbm, src_h, dst_h, attr_h, zeros_h, ones_h,
              agg_out, cnt_out,
              acc, es, ed, ea, ig, isx, gbuf, zbuf, dbuf, obuf,
              sem_i, sem_g, sem_s):
    core = lax.axis_index("c")
    sub = lax.axis_index("s")
    pltpu.sync_copy(zeros_h, zbuf)
    pltpu.sync_copy(ones_h, obuf)
    row0 = sub * RPT

    def zero_acc():
        for z in range(RPT // RZ):
            pltpu.sync_copy(zbuf, acc.at[pl.ds(row0 + z * RZ, RZ)])

    def dump_to(out_slab):
        for z in range(RPT // RZ):
            rr = row0 + z * RZ
            pltpu.sync_copy(acc.at[pl.ds(rr, RZ)], dbuf)
            pltpu.sync_copy(dbuf, out_slab.at[pl.ds(rr, RZ)])

    def make_pair_body(chunk, base, feature):
        def pair_body(t, carry):
            ldescs = []
            for h in range(2):
                b = base + (2 * t + h) * FILL1
                ldescs.append(pltpu.async_copy(
                    dst_h.at[pl.ds(b, FILL1)], ed.at[h], sem_i))
                ldescs.append(pltpu.async_copy(
                    attr_h.at[pl.ds(b, FILL1)], ea.at[h], sem_i))
                if feature:
                    ldescs.append(pltpu.async_copy(
                        src_h.at[pl.ds(b, FILL1)], es.at[h], sem_i))
            for d in ldescs:
                d.wait()
            for h in range(2):
                for j in range(SUP1):
                    for i in range(EB // 16):
                        sl = pl.ds(i * 16, 16)
                        slf = pl.ds(j * EB + i * 16, 16)
                        a16 = ea[h, slf]
                        if feature:
                            ig[h, j, sl] = a16 * (8 * N) + es[h, slf] * 8 + chunk
                        isx[h, j, sl] = a16 * NP + ed[h, slf]
            if feature:
                gd = []
                for h in range(2):
                    for j in range(SUP1):
                        gd.append(pltpu.async_copy(
                            xp_hbm.at[ig.at[h, j]], gbuf.at[h, j], sem_g))
                sd = []
                for d in gd[:SUP1]:
                    d.wait()
                for j in range(SUP1):
                    sd.append(pltpu.async_copy(
                        gbuf.at[0, j], acc.at[isx.at[0, j]], sem_s, add=True))
                for d in gd[SUP1:]:
                    d.wait()
                for j in range(SUP1):
                    sd.append(pltpu.async_copy(
                        gbuf.at[1, j], acc.at[isx.at[1, j]], sem_s, add=True))
            else:
                sd = []
                for h in range(2):
                    for j in range(SUP1):
                        sd.append(pltpu.async_copy(
                            obuf, acc.at[isx.at[h, j]], sem_s, add=True))
            for d in sd:
                d.wait()
            return carry
        return pair_body

    ebase = sub * EPT
    for r in range(4):
        chunk = core * 4 + r
        zero_acc()
        plsc.subcore_barrier()
        lax.fori_loop(0, NP1, make_pair_body(chunk, ebase, True), 0)
        plsc.subcore_barrier()
        dump_to(agg_out.at[chunk])
        plsc.subcore_barrier()

    zero_acc()
    plsc.subcore_barrier()
    cbase = core * (EPAD // 2) + sub * EPT2
    lax.fori_loop(0, NPC, make_pair_body(0, cbase, False), 0)
    plsc.subcore_barrier()
    dump_to(cnt_out.at[core])


_edge1_call = pl.kernel(
    _sc_edge1,
    out_type=[
        jax.ShapeDtypeStruct((8, R1, 16), _f32),
        jax.ShapeDtypeStruct((2, R1, 16), _f32),
    ],
    mesh=_MESH,
    compiler_params=pltpu.CompilerParams(use_tc_tiling_on_sc=False),
    scratch_types=[
        pltpu.VMEM_SHARED((R1, 16), _f32),
        pltpu.VMEM((2, FILL1), jnp.int32),
        pltpu.VMEM((2, FILL1), jnp.int32),
        pltpu.VMEM((2, FILL1), jnp.int32),
        pltpu.VMEM((2, SUP1, EB), jnp.int32),
        pltpu.VMEM((2, SUP1, EB), jnp.int32),
        pltpu.VMEM((2, SUP1, EB, 16), _f32),
        pltpu.VMEM((RZ, 16), _f32),
        pltpu.VMEM((RZ, 16), _f32),
        pltpu.VMEM((EB, 16), _f32),
        pltpu.SemaphoreType.DMA,
        pltpu.SemaphoreType.DMA,
        pltpu.SemaphoreType.DMA,
    ],
)


def _mid(a_ref, c_ref, xr_ref, sc_ref, sh_ref, wl_ref, wr_ref, b2_ref,
         hp_ref, hr_ref, c2_ref):
    j = pl.program_id(1)
    a = jnp.concatenate([a_ref[k, 0] for k in range(8)], axis=-1)
    c = c_ref[0, 0] + c_ref[1, 0]
    mean = a / jnp.maximum(c[:, :1], 1.0)
    t = jnp.maximum(mean + xr_ref[0], 0.0)
    hb = t * sc_ref[0] + sh_ref[0]
    pj = jnp.dot(hb, wl_ref[0], preferred_element_type=_f32)
    rj = jnp.dot(hb, wr_ref[0], preferred_element_type=_f32)

    @pl.when(j == 0)
    def _():
        hp_ref[...] = pj
        hr_ref[...] = rj + b2_ref[...]
        c2_ref[...] = c

    @pl.when(j != 0)
    def _():
        hp_ref[...] += pj
        hr_ref[...] += rj
        c2_ref[...] += c


_mid_call = pl.pallas_call(
    _mid,
    grid=(NBLK, NREL),
    in_specs=[
        pl.BlockSpec((8, 1, BLK, 16), lambda i, j: (0, j, i, 0)),
        pl.BlockSpec((2, 1, BLK, 16), lambda i, j: (0, j, i, 0)),
        pl.BlockSpec((1, BLK, 128), lambda i, j: (j, i, 0)),
        pl.BlockSpec((1, 1, 128), lambda i, j: (j, 0, 0)),
        pl.BlockSpec((1, 1, 128), lambda i, j: (j, 0, 0)),
        pl.BlockSpec((1, 128, 128), lambda i, j: (j, 0, 0)),
        pl.BlockSpec((1, 128, 128), lambda i, j: (j, 0, 0)),
        pl.BlockSpec((1, 128), lambda i, j: (0, 0)),
    ],
    out_specs=[
        pl.BlockSpec((BLK, 128), lambda i, j: (i, 0)),
        pl.BlockSpec((BLK, 128), lambda i, j: (i, 0)),
        pl.BlockSpec((BLK, 16), lambda i, j: (i, 0)),
    ],
    out_shape=[
        jax.ShapeDtypeStruct((N, 128), _f32),
        jax.ShapeDtypeStruct((N, 128), _f32),
        jax.ShapeDtypeStruct((N, 16), _f32),
    ],
)


def _sc_edge2(hp_hbm, src_h, dst_h, zeros_h, agg_out,
              acc, es, ed, ig, isx, gbuf, zbuf, dbuf,
              sem_i, sem_g, sem_s):
    core = lax.axis_index("c")
    sub = lax.axis_index("s")
    pltpu.sync_copy(zeros_h, zbuf)
    row0 = sub * (R2 // NS)
    ebase = sub * EPT
    for r in range(2):
        chunk = core * 2 + r
        pltpu.sync_copy(zbuf, acc.at[pl.ds(row0, RZ)])
        plsc.subcore_barrier()

        def pair_body(t, carry):
            ldescs = []
            for h in range(2):
                b = ebase + (2 * t + h) * FILL2
                ldescs.append(pltpu.async_copy(
                    src_h.at[pl.ds(b, FILL2)], es.at[h], sem_i))
                ldescs.append(pltpu.async_copy(
                    dst_h.at[pl.ds(b, FILL2)], ed.at[h], sem_i))
            for d in ldescs:
                d.wait()
            for h in range(2):
                for j in range(SUP2):
                    for i in range(EB // 16):
                        sl = pl.ds(i * 16, 16)
                        slf = pl.ds(j * EB + i * 16, 16)
                        ig[h, j, sl] = es[h, slf] * 4 + chunk
                        isx[h, j, sl] = ed[h, slf]
            gd = []
            for h in range(2):
                for j in range(SUP2):
                    gd.append(pltpu.async_copy(
                        hp_hbm.at[ig.at[h, j]], gbuf.at[h, j], sem_g))
            sd = []
            for d in gd[:SUP2]:
                d.wait()
            for j in range(SUP2):
                sd.append(pltpu.async_copy(
                    gbuf.at[0, j], acc.at[isx.at[0, j]], sem_s, add=True))
            for d in gd[SUP2:]:
                d.wait()
            for j in range(SUP2):
                sd.append(pltpu.async_copy(
                    gbuf.at[1, j], acc.at[isx.at[1, j]], sem_s, add=True))
            for d in sd:
                d.wait()
            return carry

        lax.fori_loop(0, NP2B, pair_body, 0)
        plsc.subcore_barrier()
        pltpu.sync_copy(acc.at[pl.ds(row0, RZ)], dbuf)
        pltpu.sync_copy(dbuf, agg_out.at[chunk, pl.ds(row0, RZ)])
        plsc.subcore_barrier()


_edge2_call = pl.kernel(
    _sc_edge2,
    out_type=jax.ShapeDtypeStruct((4, R2, 32), _f32),
    mesh=_MESH,
    compiler_params=pltpu.CompilerParams(use_tc_tiling_on_sc=False),
    scratch_types=[
        pltpu.VMEM_SHARED((R2, 32), _f32),
        pltpu.VMEM((2, FILL2), jnp.int32),
        pltpu.VMEM((2, FILL2), jnp.int32),
        pltpu.VMEM((2, SUP2, EB), jnp.int32),
        pltpu.VMEM((2, SUP2, EB), jnp.int32),
        pltpu.VMEM((2, SUP2, EB, 32), _f32),
        pltpu.VMEM((RZ, 32), _f32),
        pltpu.VMEM((RZ, 32), _f32),
        pltpu.SemaphoreType.DMA,
        pltpu.SemaphoreType.DMA,
        pltpu.SemaphoreType.DMA,
    ],
)


def _fin(a_ref, c_ref, hr_ref, o_ref):
    a = jnp.concatenate([a_ref[k] for k in range(4)], axis=-1)
    o_ref[...] = a / jnp.maximum(c_ref[:, :1], 1.0) + hr_ref[...]


_fin_call = pl.pallas_call(
    _fin,
    grid=(NBLK,),
    in_specs=[
        pl.BlockSpec((4, BLK, 32), lambda i: (0, i, 0)),
        pl.BlockSpec((BLK, 16), lambda i: (i, 0)),
        pl.BlockSpec((BLK, 128), lambda i: (i, 0)),
    ],
    out_specs=pl.BlockSpec((BLK, 128), lambda i: (i, 0)),
    out_shape=jax.ShapeDtypeStruct((N, 128), _f32),
)


def kernel(x, edge_index, edge_attr, params):
    names = ('gd', 'gg', 'bg', 'gp', 'dp')
    Wl = jnp.concatenate([params[n]['Wl'] for n in names], 1)
    Wr = jnp.concatenate([params[n]['Wr'] for n in names], 1)
    b = jnp.concatenate([params[n]['b'] for n in names], 0)
    Wl5 = Wl.reshape(128, NREL, 128).transpose(1, 0, 2)
    Wr5 = Wr.reshape(128, NREL, 128).transpose(1, 0, 2)
    Wcat = jnp.concatenate([Wl5, Wr5], axis=2)
    bias = jnp.concatenate(
        [jnp.zeros((NREL, 1, 128), _f32), b.reshape(NREL, 1, 128)], axis=2)

    src = edge_index[0].astype(jnp.int32)
    dst = edge_index[1].astype(jnp.int32)
    attr = edge_attr.astype(jnp.int32)
    pad = EPAD - E
    srcP = jnp.concatenate([src, jnp.full((pad,), N - 1, jnp.int32)])
    dstP = jnp.concatenate([dst, jnp.full((pad,), N, jnp.int32)])
    attrP = jnp.concatenate([attr, jnp.full((pad,), NREL - 1, jnp.int32)])

    xp_t, xr_t = _mm1_call(x, Wcat, bias)
    xp_flat = xp_t.reshape(NREL * N * 8, 16)

    zeros16 = jnp.zeros((RZ, 16), _f32)
    ones16 = jnp.ones((EB, 16), _f32)
    zeros32 = jnp.zeros((RZ, 32), _f32)
    agg1c, cntc = _edge1_call(xp_flat, srcP, dstP, attrP, zeros16, ones16)
    agg1v = agg1c.reshape(8, NREL, NP, 16)
    cntv = cntc.reshape(2, NREL, NP, 16)

    bn = params['bn']
    scale = (bn['gamma'] / jnp.sqrt(bn['var'] + 1e-5))
    shift = bn['beta'] - bn['mean'] * scale
    scale5 = scale.reshape(NREL, 1, 128)
    shift5 = shift.reshape(NREL, 1, 128)
    Wl2 = jnp.concatenate(
        [params['mu']['Wl'], params['logvar']['Wl']], 1).reshape(NREL, 128, 128)
    Wr2 = jnp.concatenate(
        [params['mu']['Wr'], params['logvar']['Wr']], 1).reshape(NREL, 128, 128)
    b2 = jnp.concatenate(
        [params['mu']['b'], params['logvar']['b']], 0).reshape(1, 128)

    hp, hr, cnt2 = _mid_call(agg1v, cntv, xr_t, scale5, shift5,
                             Wl2, Wr2, b2)
    hp_flat = hp.reshape(4 * N, 32)

    agg2c = _edge2_call(hp_flat, srcP, dstP, zeros32)

    out = _fin_call(agg2c, cnt2, hr)
    return out[:, :64], out[:, 64:]
